# baseline (speedup 1.0000x reference)
import jax
import jax.numpy as jnp
from jax import lax
from jax.experimental import pallas as pl
from jax.experimental.pallas import tpu as pltpu

NUM_OBJ = 10
D = 128
TB = 64
N = TB * NUM_OBJ
CH = 8
NC = CH * NUM_OBJ


def _ln(x, g, b):
    m = jnp.mean(x, axis=-1, keepdims=True)
    xc = x - m
    v = jnp.mean(xc * xc, axis=-1, keepdims=True)
    return xc * lax.rsqrt(v + 1e-5) * g + b


def _fused(x_ref, a_ref, we1_ref, be1_ref, we2_ref, be2_ref,
           ge_ref, bbe_ref, we3_ref, be3_ref, wn1_ref,
           bn1_ref, wn2_ref, bn2_ref, gn_ref, bbn_ref, wn3_ref, bn3_ref,
           o_ref):
    we1a = we1_ref[0:D, :]
    we1b = we1_ref[D:2 * D, :]
    be1 = be1_ref[...]
    we2 = we2_ref[...]
    be2 = be2_ref[...]
    gs = ge_ref[...] * jnp.sqrt(jnp.float32(D))
    bbe = bbe_ref[...]
    we3 = we3_ref[...]
    be3_9 = (NUM_OBJ - 1) * be3_ref[...]
    wn1x = wn1_ref[0:D, :]
    wn1g = wn1_ref[D + 4:D + 4 + D, :]
    wn1a = [wn1_ref[D + k:D + k + 1, :] for k in range(4)]
    bn1 = bn1_ref[...]
    wn2 = wn2_ref[...]
    bn2 = bn2_ref[...]
    gn = gn_ref[...]
    bbn = bbn_ref[...]
    wn3 = wn3_ref[...]
    bn3 = bn3_ref[...]

    a = a_ref[...].astype(jnp.float32)
    i_col = lax.broadcasted_iota(jnp.int32, (NC, 1), 0) % NUM_OBJ
    i_colf = i_col.astype(jnp.float32)
    rep0 = lax.broadcasted_iota(jnp.int32, (NC, TB), 0) // NUM_OBJ
    rep1 = lax.broadcasted_iota(jnp.int32, (NC, TB), 1)

    def chunk(c, carry):
        Xc = x_ref[pl.ds(c * CH, CH), :, :].reshape(NC, D)
        P = jnp.dot(Xc, we1a) + be1
        Q = jnp.dot(Xc, we1b)
        Qd = pltpu.roll(Q, NUM_OBJ, 0)
        S = jnp.zeros((NC, D), jnp.float32)
        for s in range(1, NUM_OBJ):
            pre = jnp.where(i_col >= s, Q, Qd)
            Qr = pltpu.roll(pre, NC - s, 0)
            e1 = jax.nn.relu(P + Qr)
            e = jnp.dot(e1, we2) + be2
            xc = e - jnp.mean(e, axis=-1, keepdims=True)
            rs = lax.rsqrt(jnp.sum(xc * xc, axis=-1, keepdims=True)
                           + jnp.float32(D * 1e-5))
            S = S + jax.nn.relu(xc * rs * gs + bbe)
        agg = jnp.dot(S, we3) + be3_9
        rep = (rep0 + c * CH == rep1).astype(jnp.float32)
        cc = jnp.dot(rep, a) - 4.0 * i_colf
        actc = jnp.zeros((NC, D), jnp.float32)
        for k in range(4):
            actc = actc + jnp.where(cc == float(k), wn1a[k], 0.0)
        h = jax.nn.relu(jnp.dot(Xc, wn1x) + actc + jnp.dot(agg, wn1g) + bn1)
        h = jnp.dot(h, wn2) + bn2
        h = jax.nn.relu(_ln(h, gn, bbn))
        out_c = jnp.dot(h, wn3) + bn3
        o_ref[pl.ds(c * CH, CH), :, :] = out_c.reshape(CH, NUM_OBJ, D)
        return carry

    lax.fori_loop(0, TB // CH, chunk, 0)


def kernel(states, We1, be1, We2, be2, ge, bbe, We3, be3,
           Wn1, bn1, Wn2, bn2, gn, bbn, Wn3, bn3, action):
    B, n, d = states.shape
    act2 = action.reshape(B, 1)
    row2 = lambda v: v.reshape(1, -1)
    full = lambda shp: pl.BlockSpec(shp, lambda b: (0, 0))
    out = pl.pallas_call(
        _fused,
        grid=(B // TB,),
        in_specs=[
            pl.BlockSpec((TB, NUM_OBJ, d), lambda b: (b, 0, 0)),
            pl.BlockSpec((TB, 1), lambda b: (b, 0)),
            full((2 * d, d)), full((1, d)),
            full((d, d)), full((1, d)),
            full((1, d)), full((1, d)),
            full((d, d)), full((1, d)),
            full((d + 4 + d, d)),
            full((1, d)),
            full((d, d)), full((1, d)),
            full((1, d)), full((1, d)),
            full((d, d)), full((1, d)),
        ],
        out_specs=pl.BlockSpec((TB, NUM_OBJ, d), lambda b: (b, 0, 0)),
        out_shape=jax.ShapeDtypeStruct((B, n, d), jnp.float32),
        compiler_params=pltpu.CompilerParams(
            dimension_semantics=("parallel",)),
    )(states, act2,
      We1, row2(be1),
      We2, row2(be2), row2(ge), row2(bbe),
      We3, row2(be3),
      Wn1, row2(bn1),
      Wn2, row2(bn2), row2(gn), row2(bbn),
      Wn3, row2(bn3))
    return out


# speedup vs baseline: 2.5693x; 2.5693x over previous
import jax
import jax.numpy as jnp
from jax import lax
from jax.experimental import pallas as pl
from jax.experimental.pallas import tpu as pltpu

NUM_OBJ = 10
D = 128
TB = 128
N = TB * NUM_OBJ


def _ln(x, g, b):
    m = jnp.mean(x, axis=-1, keepdims=True)
    xc = x - m
    v = jnp.mean(xc * xc, axis=-1, keepdims=True)
    return xc * lax.rsqrt(v + 1e-5) * g + b


def _fused(x_ref, a_ref, we1_ref, be1_ref, we2_ref, be2_ref,
           ge_ref, bbe_ref, we3_ref, be3_ref, wn1_ref,
           bn1_ref, wn2_ref, bn2_ref, gn_ref, bbn_ref, wn3_ref, bn3_ref,
           o_ref):
    X = x_ref[...].reshape(N, D)
    P = jnp.dot(X, we1_ref[0:D, :]) + be1_ref[...]
    Q = jnp.dot(X, we1_ref[D:2 * D, :])
    we2 = we2_ref[...]
    be2 = be2_ref[...]
    gs = ge_ref[...] * jnp.sqrt(jnp.float32(D))
    bbe = bbe_ref[...]
    i_col = lax.broadcasted_iota(jnp.int32, (N, 1), 0) % NUM_OBJ
    Qd = pltpu.roll(Q, NUM_OBJ, 0)
    S = jnp.zeros((N, D), jnp.float32)
    for s in range(1, NUM_OBJ):
        pre = jnp.where(i_col >= s, Q, Qd)
        Qr = pltpu.roll(pre, N - s, 0)
        e1 = jax.nn.relu(P + Qr)
        e = jnp.dot(e1, we2) + be2
        xc = e - jnp.mean(e, axis=-1, keepdims=True)
        rs = lax.rsqrt(jnp.sum(xc * xc, axis=-1, keepdims=True)
                       + jnp.float32(D * 1e-5))
        S = S + jax.nn.relu(xc * rs * gs + bbe)
    agg = jnp.dot(S, we3_ref[...]) + (NUM_OBJ - 1) * be3_ref[...]
    a = a_ref[...].astype(jnp.float32)
    rep = (lax.broadcasted_iota(jnp.int32, (N, TB), 0) // NUM_OBJ
           == lax.broadcasted_iota(jnp.int32, (N, TB), 1))
    arow = jnp.dot(rep.astype(jnp.float32), a)
    c = arow - 4.0 * i_col.astype(jnp.float32)
    actc = jnp.zeros_like(X)
    for k in range(4):
        actc = actc + jnp.where(c == float(k), wn1_ref[D + k:D + k + 1, :], 0.0)
    h = (jnp.dot(X, wn1_ref[0:D, :]) + actc
         + jnp.dot(agg, wn1_ref[D + 4:D + 4 + D, :]) + bn1_ref[...])
    h = jax.nn.relu(h)
    h = jnp.dot(h, wn2_ref[...]) + bn2_ref[...]
    h = jax.nn.relu(_ln(h, gn_ref[...], bbn_ref[...]))
    o_ref[...] = (jnp.dot(h, wn3_ref[...]) + bn3_ref[...]).reshape(TB, NUM_OBJ, D)


def kernel(states, We1, be1, We2, be2, ge, bbe, We3, be3,
           Wn1, bn1, Wn2, bn2, gn, bbn, Wn3, bn3, action):
    B, n, d = states.shape
    act2 = action.reshape(B, 1)
    row2 = lambda v: v.reshape(1, -1)
    full = lambda shp: pl.BlockSpec(shp, lambda b: (0, 0))
    out = pl.pallas_call(
        _fused,
        grid=(B // TB,),
        in_specs=[
            pl.BlockSpec((TB, NUM_OBJ, d), lambda b: (b, 0, 0)),
            pl.BlockSpec((TB, 1), lambda b: (b, 0)),
            full((2 * d, d)), full((1, d)),
            full((d, d)), full((1, d)),
            full((1, d)), full((1, d)),
            full((d, d)), full((1, d)),
            full((d + 4 + d, d)),
            full((1, d)),
            full((d, d)), full((1, d)),
            full((1, d)), full((1, d)),
            full((d, d)), full((1, d)),
        ],
        out_specs=pl.BlockSpec((TB, NUM_OBJ, d), lambda b: (b, 0, 0)),
        out_shape=jax.ShapeDtypeStruct((B, n, d), jnp.float32),
        compiler_params=pltpu.CompilerParams(
            dimension_semantics=("parallel",)),
    )(states, act2,
      We1, row2(be1),
      We2, row2(be2), row2(ge), row2(bbe),
      We3, row2(be3),
      Wn1, row2(bn1),
      Wn2, row2(bn2), row2(gn), row2(bbn),
      Wn3, row2(bn3))
    return out


# speedup vs baseline: 3.6088x; 1.4046x over previous
import jax
import jax.numpy as jnp
from jax import lax
from jax.experimental import pallas as pl
from jax.experimental.pallas import tpu as pltpu

NUM_OBJ = 10
D = 128
TB = 128
N = TB * NUM_OBJ


def _ln(x, g, b):
    m = jnp.mean(x, axis=-1, keepdims=True)
    xc = x - m
    v = jnp.mean(xc * xc, axis=-1, keepdims=True)
    return xc * lax.rsqrt(v + 1e-5) * g + b


def _fused(x_ref, a_ref, we1_ref, be1_ref, we2_ref, be2_ref,
           ge_ref, bbe_ref, we3_ref, be3_ref, wn1_ref,
           bn1_ref, wn2_ref, bn2_ref, gn_ref, bbn_ref, wn3_ref, bn3_ref,
           o_ref):
    X = x_ref[...].reshape(N, D)
    P = jnp.dot(X, we1_ref[0:D, :]) + be1_ref[...]
    Q = jnp.dot(X, we1_ref[D:2 * D, :])
    we2 = we2_ref[...]
    be2 = be2_ref[...]
    gs = ge_ref[...] * jnp.sqrt(jnp.float32(D))
    bbe = bbe_ref[...]
    S = jnp.zeros((N, D), jnp.float32)
    for s in range(1, NUM_OBJ):
        Qr = pltpu.roll(Q, N - s * TB, 0)
        e1 = jax.nn.relu(P + Qr)
        e = jnp.dot(e1, we2) + be2
        xc = e - jnp.mean(e, axis=-1, keepdims=True)
        rs = lax.rsqrt(jnp.sum(xc * xc, axis=-1, keepdims=True)
                       + jnp.float32(D * 1e-5))
        S = S + jax.nn.relu(xc * rs * gs + bbe)
    agg = jnp.dot(S, we3_ref[...]) + (NUM_OBJ - 1) * be3_ref[...]
    a = a_ref[...].astype(jnp.float32)
    rep = (lax.broadcasted_iota(jnp.int32, (N, TB), 0) % TB
           == lax.broadcasted_iota(jnp.int32, (N, TB), 1))
    arow = jnp.dot(rep.astype(jnp.float32), a)
    i_col = lax.broadcasted_iota(jnp.int32, (N, 1), 0) // TB
    c = arow - 4.0 * i_col.astype(jnp.float32)
    actc = jnp.zeros_like(X)
    for k in range(4):
        actc = actc + jnp.where(c == float(k), wn1_ref[D + k:D + k + 1, :], 0.0)
    h = (jnp.dot(X, wn1_ref[0:D, :]) + actc
         + jnp.dot(agg, wn1_ref[D + 4:D + 4 + D, :]) + bn1_ref[...])
    h = jax.nn.relu(h)
    h = jnp.dot(h, wn2_ref[...]) + bn2_ref[...]
    h = jax.nn.relu(_ln(h, gn_ref[...], bbn_ref[...]))
    o_ref[...] = (jnp.dot(h, wn3_ref[...]) + bn3_ref[...]).reshape(NUM_OBJ, TB, D)


def kernel(states, We1, be1, We2, be2, ge, bbe, We3, be3,
           Wn1, bn1, Wn2, bn2, gn, bbn, Wn3, bn3, action):
    B, n, d = states.shape
    states_t = jnp.transpose(states, (1, 0, 2))
    act2 = action.reshape(B, 1)
    row2 = lambda v: v.reshape(1, -1)
    full = lambda shp: pl.BlockSpec(shp, lambda b: (0, 0))
    out = pl.pallas_call(
        _fused,
        grid=(B // TB,),
        in_specs=[
            pl.BlockSpec((NUM_OBJ, TB, d), lambda b: (0, b, 0)),
            pl.BlockSpec((TB, 1), lambda b: (b, 0)),
            full((2 * d, d)), full((1, d)),
            full((d, d)), full((1, d)),
            full((1, d)), full((1, d)),
            full((d, d)), full((1, d)),
            full((d + 4 + d, d)),
            full((1, d)),
            full((d, d)), full((1, d)),
            full((1, d)), full((1, d)),
            full((d, d)), full((1, d)),
        ],
        out_specs=pl.BlockSpec((NUM_OBJ, TB, d), lambda b: (0, b, 0)),
        out_shape=jax.ShapeDtypeStruct((n, B, d), jnp.float32),
        compiler_params=pltpu.CompilerParams(
            dimension_semantics=("parallel",)),
    )(states_t, act2,
      We1, row2(be1),
      We2, row2(be2), row2(ge), row2(bbe),
      We3, row2(be3),
      Wn1, row2(bn1),
      Wn2, row2(bn2), row2(gn), row2(bbn),
      Wn3, row2(bn3))
    return jnp.transpose(out, (1, 0, 2))


# speedup vs baseline: 4.0185x; 1.1135x over previous
import jax
import jax.numpy as jnp
from jax import lax
from jax.experimental import pallas as pl
from jax.experimental.pallas import tpu as pltpu

NUM_OBJ = 10
D = 128
TB = 256
N = TB * NUM_OBJ


def _ln(x, g, b):
    m = jnp.mean(x, axis=-1, keepdims=True)
    xc = x - m
    v = jnp.mean(xc * xc, axis=-1, keepdims=True)
    return xc * lax.rsqrt(v + 1e-5) * g + b


def _fused(x_ref, a_ref, we1_ref, be1_ref, we2_ref, be2_ref,
           ge_ref, bbe_ref, we3_ref, be3_ref, wn1_ref,
           bn1_ref, wn2_ref, bn2_ref, gn_ref, bbn_ref, wn3_ref, bn3_ref,
           o_ref):
    X = x_ref[...].reshape(N, D)
    P = jnp.dot(X, we1_ref[0:D, :]) + be1_ref[...]
    Q = jnp.dot(X, we1_ref[D:2 * D, :])
    we2 = we2_ref[...]
    be2 = be2_ref[...]
    gs = ge_ref[...] * jnp.sqrt(jnp.float32(D))
    bbe = bbe_ref[...]
    S = jnp.zeros((N, D), jnp.float32)
    for s in range(1, NUM_OBJ):
        Qr = pltpu.roll(Q, N - s * TB, 0)
        e1 = jax.nn.relu(P + Qr)
        e = jnp.dot(e1, we2) + be2
        xc = e - jnp.mean(e, axis=-1, keepdims=True)
        rs = lax.rsqrt(jnp.sum(xc * xc, axis=-1, keepdims=True)
                       + jnp.float32(D * 1e-5))
        S = S + jax.nn.relu(xc * rs * gs + bbe)
    agg = jnp.dot(S, we3_ref[...]) + (NUM_OBJ - 1) * be3_ref[...]
    a = a_ref[...].astype(jnp.float32)
    rep = (lax.broadcasted_iota(jnp.int32, (N, TB), 0) % TB
           == lax.broadcasted_iota(jnp.int32, (N, TB), 1))
    arow = jnp.dot(rep.astype(jnp.float32), a)
    i_col = lax.broadcasted_iota(jnp.int32, (N, 1), 0) // TB
    c = arow - 4.0 * i_col.astype(jnp.float32)
    actc = jnp.zeros_like(X)
    for k in range(4):
        actc = actc + jnp.where(c == float(k), wn1_ref[D + k:D + k + 1, :], 0.0)
    h = (jnp.dot(X, wn1_ref[0:D, :]) + actc
         + jnp.dot(agg, wn1_ref[D + 4:D + 4 + D, :]) + bn1_ref[...])
    h = jax.nn.relu(h)
    h = jnp.dot(h, wn2_ref[...]) + bn2_ref[...]
    h = jax.nn.relu(_ln(h, gn_ref[...], bbn_ref[...]))
    o_ref[...] = (jnp.dot(h, wn3_ref[...]) + bn3_ref[...]).reshape(NUM_OBJ, TB, D)


def kernel(states, We1, be1, We2, be2, ge, bbe, We3, be3,
           Wn1, bn1, Wn2, bn2, gn, bbn, Wn3, bn3, action):
    B, n, d = states.shape
    states_t = jnp.transpose(states, (1, 0, 2))
    act2 = action.reshape(B, 1)
    row2 = lambda v: v.reshape(1, -1)
    full = lambda shp: pl.BlockSpec(shp, lambda b: (0, 0))
    out = pl.pallas_call(
        _fused,
        grid=(B // TB,),
        in_specs=[
            pl.BlockSpec((NUM_OBJ, TB, d), lambda b: (0, b, 0)),
            pl.BlockSpec((TB, 1), lambda b: (b, 0)),
            full((2 * d, d)), full((1, d)),
            full((d, d)), full((1, d)),
            full((1, d)), full((1, d)),
            full((d, d)), full((1, d)),
            full((d + 4 + d, d)),
            full((1, d)),
            full((d, d)), full((1, d)),
            full((1, d)), full((1, d)),
            full((d, d)), full((1, d)),
        ],
        out_specs=pl.BlockSpec((NUM_OBJ, TB, d), lambda b: (0, b, 0)),
        out_shape=jax.ShapeDtypeStruct((n, B, d), jnp.float32),
        compiler_params=pltpu.CompilerParams(
            dimension_semantics=("parallel",)),
    )(states_t, act2,
      We1, row2(be1),
      We2, row2(be2), row2(ge), row2(bbe),
      We3, row2(be3),
      Wn1, row2(bn1),
      Wn2, row2(bn2), row2(gn), row2(bbn),
      Wn3, row2(bn3))
    return jnp.transpose(out, (1, 0, 2))
